# baseline (speedup 1.0000x reference)
import functools

import jax
import jax.numpy as jnp
from jax import lax
from jax.experimental import pallas as pl
from jax.experimental.pallas import tpu as pltpu
from jax.experimental.pallas import tpu_sc as plsc

N = 10000
E = 160000
D = 256
DW = 128
N_PAD = 10240
E_PAD = 163840
NHALF = N_PAD // 2
ACC_ROWS = NHALF + 8
CHUNK = 128
EDGES_PER_TILE = E_PAD // 16
CHUNKS_PER_TILE = EDGES_PER_TILE // CHUNK
ROWS_PER_TILE = NHALF // 16
DEG_W = 8
DEG_CHUNK = 128
DEG_CHUNKS = EDGES_PER_TILE // DEG_CHUNK

_mesh = plsc.VectorSubcoreMesh(core_axis_name="c", subcore_axis_name="s")


@functools.partial(
    pl.kernel,
    out_type=jax.ShapeDtypeStruct((N_PAD, DEG_W), jnp.float32),
    mesh=_mesh,
    scratch_types=[
        pltpu.VMEM((EDGES_PER_TILE,), jnp.int32),
        pltpu.VMEM((DEG_CHUNKS, DEG_CHUNK), jnp.int32),
        pltpu.VMEM((DEG_CHUNK, DEG_W), jnp.float32),
        pltpu.VMEM((ROWS_PER_TILE, DEG_W), jnp.float32),
        pltpu.VMEM_SHARED((ACC_ROWS, DEG_W), jnp.float32),
    ],
)
def _deg_kernel(dst_hbm, deg_hbm, dst_v, loc_v, ones_v, zero_v, acc):
    c = lax.axis_index("c")
    s = lax.axis_index("s")
    node_base = c * NHALF
    edge_base = s * EDGES_PER_TILE

    pltpu.sync_copy(dst_hbm.at[pl.ds(edge_base, EDGES_PER_TILE)], dst_v)

    @pl.loop(0, DEG_CHUNK)
    def _ones(i):
        ones_v[i] = jnp.full((DEG_W,), 1.0, jnp.float32)

    @pl.loop(0, ROWS_PER_TILE)
    def _zeros(i):
        zero_v[i] = jnp.zeros((DEG_W,), jnp.float32)

    pltpu.sync_copy(zero_v, acc.at[pl.ds(s * ROWS_PER_TILE, ROWS_PER_TILE)])

    @pl.loop(0, DEG_CHUNKS)
    def _idx(j):
        @pl.loop(0, DEG_CHUNK // 16)
        def _idx16(i):
            v = dst_v[pl.ds(j * DEG_CHUNK + i * 16, 16)]
            loc = v - node_base
            ok = (loc >= 0) & (loc < NHALF)
            loc_v[j, pl.ds(i * 16, 16)] = jnp.where(ok, loc, NHALF)

    plsc.subcore_barrier()

    @pl.loop(0, DEG_CHUNKS)
    def _hist(j):
        pltpu.sync_copy(ones_v, acc.at[loc_v.at[j]], add=True)

    plsc.subcore_barrier()

    pltpu.sync_copy(
        acc.at[pl.ds(s * ROWS_PER_TILE, ROWS_PER_TILE)],
        deg_hbm.at[pl.ds(node_base + s * ROWS_PER_TILE, ROWS_PER_TILE)])


@functools.partial(
    pl.kernel,
    out_type=[jax.ShapeDtypeStruct((N_PAD, DW), jnp.float32)] * 2,
    mesh=_mesh,
    scratch_types=[
        pltpu.VMEM((EDGES_PER_TILE,), jnp.int32),
        pltpu.VMEM((EDGES_PER_TILE,), jnp.int32),
        pltpu.VMEM((CHUNKS_PER_TILE, CHUNK), jnp.int32),
        pltpu.VMEM((CHUNK, 256), jnp.float32),
        pltpu.VMEM_SHARED((ACC_ROWS, DW), jnp.float32),
        pltpu.SemaphoreType.DMA,
        pltpu.SemaphoreType.DMA,
        pltpu.SemaphoreType.DMA,
        pltpu.SemaphoreType.DMA,
        pltpu.SemaphoreType.DMA,
        pltpu.SemaphoreType.DMA,
        pltpu.SemaphoreType.DMA,
        pltpu.SemaphoreType.DMA,
    ],
)
def _agg_kernel(xf_hbm, ha_hbm, hb_hbm, src_hbm, dst_hbm, oa_hbm, ob_hbm,
                src_v, dst_v, loc_v, msg0,
                accum, g0, g1, g2, g3, s0, s1, s2, s3):
    c = lax.axis_index("c")
    s = lax.axis_index("s")
    node_base = c * NHALF
    edge_base = s * EDGES_PER_TILE
    row0 = s * ROWS_PER_TILE

    pltpu.sync_copy(src_hbm.at[pl.ds(edge_base, EDGES_PER_TILE)], src_v)
    pltpu.sync_copy(dst_hbm.at[pl.ds(edge_base, EDGES_PER_TILE)], dst_v)

    @pl.loop(0, CHUNKS_PER_TILE)
    def _idx(j):
        @pl.loop(0, CHUNK // 16)
        def _idx16(i):
            v = dst_v[pl.ds(j * CHUNK + i * 16, 16)]
            loc = v - node_base
            ok = (loc >= 0) & (loc < NHALF)
            loc_v[j, pl.ds(i * 16, 16)] = jnp.where(
                ok, loc, NHALF + (v & 7))

    NBUF = 1
    bufs = (msg0,)
    gsems = (g0,)
    ssems = (s0,)

    for h_hbm, o_hbm in ((ha_hbm, oa_hbm), (hb_hbm, ob_hbm)):
        pltpu.sync_copy(
            h_hbm.at[pl.ds(node_base + row0, ROWS_PER_TILE)],
            accum.at[pl.ds(row0, ROWS_PER_TILE)])

        plsc.subcore_barrier()

        def _gather(j, buf, sem):
            return pltpu.async_copy(
                xf_hbm.at[src_v.at[pl.ds(j * CHUNK, CHUNK)]], buf, sem)

        def _wait_gather(j, buf, sem):
            pltpu.make_async_copy(
                xf_hbm.at[src_v.at[pl.ds(j * CHUNK, CHUNK)]], buf, sem).wait()

        def _scatter(j, buf, sem):
            pass

        def _wait_scatter(j, buf, sem):
            pass

        for b in range(NBUF):
            _gather(b, bufs[b], gsems[b])

        @pl.loop(0, CHUNKS_PER_TILE, step=NBUF)
        def _main(j):
            for b in range(NBUF):
                k = j + b
                _wait_gather(k, bufs[b], gsems[b])
                _scatter(k, bufs[b], ssems[b])

            for b in range(NBUF):
                k = j + b

                @pl.when(k + NBUF < CHUNKS_PER_TILE)
                def _next():
                    _wait_scatter(k, bufs[b], ssems[b])
                    _gather(k + NBUF, bufs[b], gsems[b])

        for b in range(NBUF):
            _wait_scatter(CHUNKS_PER_TILE - NBUF + b, bufs[b], ssems[b])

        plsc.subcore_barrier()

        pltpu.sync_copy(
            accum.at[pl.ds(row0, ROWS_PER_TILE)],
            o_hbm.at[pl.ds(node_base + row0, ROWS_PER_TILE)])

        plsc.subcore_barrier()


def _mm_first_body(x_ref, w_ref, deg_ref, oa_ref, ob_ref):
    dis = lax.rsqrt(deg_ref[...] + 1.0)
    r = jnp.dot(x_ref[...], w_ref[...],
                preferred_element_type=jnp.float32) * dis
    oa_ref[...] = r[:, :DW]
    ob_ref[...] = r[:, DW:]


def _mm_mid_body(aa_ref, ab_ref, w_ref, deg_ref, b_ref, oa_ref, ob_ref):
    dis = lax.rsqrt(deg_ref[...] + 1.0)
    ha = jnp.maximum(aa_ref[...] * dis + b_ref[:, :DW], 0.0)
    hb = jnp.maximum(ab_ref[...] * dis + b_ref[:, DW:], 0.0)
    h = jnp.concatenate([ha, hb], axis=1)
    r = jnp.dot(h, w_ref[...], preferred_element_type=jnp.float32) * dis
    oa_ref[...] = r[:, :DW]
    ob_ref[...] = r[:, DW:]


def _epi_body(aa_ref, ab_ref, deg_ref, b_ref, o_ref):
    dis = lax.rsqrt(deg_ref[...] + 1.0)
    o_ref[:, :DW] = aa_ref[...] * dis + b_ref[:, :DW]
    o_ref[:, DW:] = ab_ref[...] * dis + b_ref[:, DW:]


_ROWS_BLK = 256
_GRID = (N_PAD // _ROWS_BLK,)
_xspec = pl.BlockSpec((_ROWS_BLK, D), lambda i: (i, 0))
_hspec = pl.BlockSpec((_ROWS_BLK, DW), lambda i: (i, 0))
_wspec = pl.BlockSpec((D, D), lambda i: (0, 0))
_dspec = pl.BlockSpec((_ROWS_BLK, 1), lambda i: (i, 0))
_bspec = pl.BlockSpec((1, D), lambda i: (0, 0))
_oshape = jax.ShapeDtypeStruct((N_PAD, D), jnp.float32)
_hshape = jax.ShapeDtypeStruct((N_PAD, DW), jnp.float32)

_mm_first = pl.pallas_call(
    _mm_first_body, grid=_GRID,
    in_specs=[_xspec, _wspec, _dspec],
    out_specs=[_hspec, _hspec], out_shape=[_hshape, _hshape])

_mm_mid = pl.pallas_call(
    _mm_mid_body, grid=_GRID,
    in_specs=[_hspec, _hspec, _wspec, _dspec, _bspec],
    out_specs=[_hspec, _hspec], out_shape=[_hshape, _hshape])

_epi = pl.pallas_call(
    _epi_body, grid=_GRID,
    in_specs=[_hspec, _hspec, _dspec, _bspec],
    out_specs=_xspec, out_shape=_oshape)


def kernel(x, edge_index, W1, b1, W2, b2, W3, b3):
    src = edge_index[0]
    dst = edge_index[1]
    pad = E_PAD - E
    src_p = jnp.concatenate([src, jnp.zeros((pad,), src.dtype)])
    dst_p = jnp.concatenate([dst, jnp.full((pad,), N_PAD, dst.dtype)])
    x_p = jnp.pad(x, ((0, N_PAD - N), (0, 0)))

    deg = _deg_kernel(dst_p)[:, :1]
    b1r = b1.reshape(1, D)
    b2r = b2.reshape(1, D)
    b3r = b3.reshape(1, D)

    ha, hb = _mm_first(x_p, W1, deg)
    aa, ab = _agg_kernel(x_p, ha, hb, src_p, dst_p)
    ha, hb = _mm_mid(aa, ab, W2, deg, b1r)
    aa, ab = _agg_kernel(x_p, ha, hb, src_p, dst_p)
    ha, hb = _mm_mid(aa, ab, W3, deg, b2r)
    aa, ab = _agg_kernel(x_p, ha, hb, src_p, dst_p)
    out = _epi(aa, ab, deg, b3r)
    return out[:N]


# speedup vs baseline: 1.0970x; 1.0970x over previous
import functools

import jax
import jax.numpy as jnp
from jax import lax
from jax.experimental import pallas as pl
from jax.experimental.pallas import tpu as pltpu
from jax.experimental.pallas import tpu_sc as plsc

N = 10000
E = 160000
D = 256
DW = 128
N_PAD = 10240
E_PAD = 163840
NHALF = N_PAD // 2
ACC_ROWS = NHALF + 8
CHUNK = 128
EDGES_PER_TILE = E_PAD // 16
CHUNKS_PER_TILE = EDGES_PER_TILE // CHUNK
ROWS_PER_TILE = NHALF // 16
DEG_W = 8
DEG_CHUNK = 128
DEG_CHUNKS = EDGES_PER_TILE // DEG_CHUNK

_mesh = plsc.VectorSubcoreMesh(core_axis_name="c", subcore_axis_name="s")


@functools.partial(
    pl.kernel,
    out_type=jax.ShapeDtypeStruct((N_PAD, DEG_W), jnp.float32),
    mesh=_mesh,
    scratch_types=[
        pltpu.VMEM((EDGES_PER_TILE,), jnp.int32),
        pltpu.VMEM((DEG_CHUNKS, DEG_CHUNK), jnp.int32),
        pltpu.VMEM((DEG_CHUNK, DEG_W), jnp.float32),
        pltpu.VMEM((ROWS_PER_TILE, DEG_W), jnp.float32),
        pltpu.VMEM_SHARED((ACC_ROWS, DEG_W), jnp.float32),
    ],
)
def _deg_kernel(dst_hbm, deg_hbm, dst_v, loc_v, ones_v, zero_v, acc):
    c = lax.axis_index("c")
    s = lax.axis_index("s")
    node_base = c * NHALF
    edge_base = s * EDGES_PER_TILE

    pltpu.sync_copy(dst_hbm.at[pl.ds(edge_base, EDGES_PER_TILE)], dst_v)

    @pl.loop(0, DEG_CHUNK)
    def _ones(i):
        ones_v[i] = jnp.full((DEG_W,), 1.0, jnp.float32)

    @pl.loop(0, ROWS_PER_TILE)
    def _zeros(i):
        zero_v[i] = jnp.zeros((DEG_W,), jnp.float32)

    pltpu.sync_copy(zero_v, acc.at[pl.ds(s * ROWS_PER_TILE, ROWS_PER_TILE)])

    @pl.loop(0, DEG_CHUNKS)
    def _idx(j):
        @pl.loop(0, DEG_CHUNK // 16)
        def _idx16(i):
            v = dst_v[pl.ds(j * DEG_CHUNK + i * 16, 16)]
            loc = v - node_base
            ok = (loc >= 0) & (loc < NHALF)
            loc_v[j, pl.ds(i * 16, 16)] = jnp.where(ok, loc, NHALF)

    plsc.subcore_barrier()

    @pl.loop(0, DEG_CHUNKS)
    def _hist(j):
        pltpu.sync_copy(ones_v, acc.at[loc_v.at[j]], add=True)

    plsc.subcore_barrier()

    pltpu.sync_copy(
        acc.at[pl.ds(s * ROWS_PER_TILE, ROWS_PER_TILE)],
        deg_hbm.at[pl.ds(node_base + s * ROWS_PER_TILE, ROWS_PER_TILE)])


@functools.partial(
    pl.kernel,
    out_type=[jax.ShapeDtypeStruct((N_PAD, DW), jnp.float32)] * 2,
    mesh=_mesh,
    scratch_types=[
        pltpu.VMEM((EDGES_PER_TILE,), jnp.int32),
        pltpu.VMEM((EDGES_PER_TILE,), jnp.int32),
        pltpu.VMEM((CHUNKS_PER_TILE, CHUNK), jnp.int32),
        pltpu.VMEM((CHUNK, DW), jnp.float32),
        pltpu.VMEM((CHUNK, DW), jnp.float32),
        pltpu.VMEM((CHUNK, DW), jnp.float32),
        pltpu.VMEM((CHUNK, DW), jnp.float32),
        pltpu.VMEM_SHARED((ACC_ROWS, DW), jnp.float32),
        pltpu.SemaphoreType.DMA,
        pltpu.SemaphoreType.DMA,
        pltpu.SemaphoreType.DMA,
        pltpu.SemaphoreType.DMA,
        pltpu.SemaphoreType.DMA,
        pltpu.SemaphoreType.DMA,
        pltpu.SemaphoreType.DMA,
        pltpu.SemaphoreType.DMA,
    ],
)
def _agg_kernel(ha_hbm, hb_hbm, src_hbm, dst_hbm, oa_hbm, ob_hbm,
                src_v, dst_v, loc_v, msg0, msg1, msg2, msg3,
                accum, g0, g1, g2, g3, s0, s1, s2, s3):
    c = lax.axis_index("c")
    s = lax.axis_index("s")
    node_base = c * NHALF
    edge_base = s * EDGES_PER_TILE
    row0 = s * ROWS_PER_TILE

    pltpu.sync_copy(src_hbm.at[pl.ds(edge_base, EDGES_PER_TILE)], src_v)
    pltpu.sync_copy(dst_hbm.at[pl.ds(edge_base, EDGES_PER_TILE)], dst_v)

    @pl.loop(0, CHUNKS_PER_TILE)
    def _idx(j):
        @pl.loop(0, CHUNK // 16)
        def _idx16(i):
            v = dst_v[pl.ds(j * CHUNK + i * 16, 16)]
            loc = v - node_base
            ok = (loc >= 0) & (loc < NHALF)
            loc_v[j, pl.ds(i * 16, 16)] = jnp.where(
                ok, loc, NHALF + (v & 7))

    NBUF = 2
    bufs = (msg0, msg1)
    gsems = (g0, g1)
    ssems = (s0, s1)

    for h_hbm, o_hbm in ((ha_hbm, oa_hbm), (hb_hbm, ob_hbm)):
        pltpu.sync_copy(
            h_hbm.at[pl.ds(node_base + row0, ROWS_PER_TILE)],
            accum.at[pl.ds(row0, ROWS_PER_TILE)])

        plsc.subcore_barrier()

        def _gather(j, buf, sem):
            return pltpu.async_copy(
                h_hbm.at[src_v.at[pl.ds(j * CHUNK, CHUNK)]], buf, sem)

        def _wait_gather(j, buf, sem):
            pltpu.make_async_copy(
                h_hbm.at[src_v.at[pl.ds(j * CHUNK, CHUNK)]], buf, sem).wait()

        def _scatter(j, buf, sem):
            pltpu.async_copy(buf, accum.at[loc_v.at[j]], sem, add=True)

        def _wait_scatter(j, buf, sem):
            pltpu.make_async_copy(buf, accum.at[loc_v.at[j]], sem).wait()

        for b in range(NBUF):
            _gather(b, bufs[b], gsems[b])

        @pl.loop(0, CHUNKS_PER_TILE, step=NBUF)
        def _main(j):
            for b in range(NBUF):
                k = j + b
                _wait_gather(k, bufs[b], gsems[b])
                _scatter(k, bufs[b], ssems[b])

            for b in range(NBUF):
                k = j + b

                @pl.when(k + NBUF < CHUNKS_PER_TILE)
                def _next():
                    _wait_scatter(k, bufs[b], ssems[b])
                    _gather(k + NBUF, bufs[b], gsems[b])

        for b in range(NBUF):
            _wait_scatter(CHUNKS_PER_TILE - NBUF + b, bufs[b], ssems[b])

        plsc.subcore_barrier()

        pltpu.sync_copy(
            accum.at[pl.ds(row0, ROWS_PER_TILE)],
            o_hbm.at[pl.ds(node_base + row0, ROWS_PER_TILE)])

        plsc.subcore_barrier()


def _mm_first_body(x_ref, w_ref, deg_ref, oa_ref, ob_ref):
    dis = lax.rsqrt(deg_ref[...] + 1.0)
    r = jnp.dot(x_ref[...], w_ref[...],
                preferred_element_type=jnp.float32) * dis
    oa_ref[...] = r[:, :DW]
    ob_ref[...] = r[:, DW:]


def _mm_mid_body(aa_ref, ab_ref, w_ref, deg_ref, b_ref, oa_ref, ob_ref):
    dis = lax.rsqrt(deg_ref[...] + 1.0)
    ha = jnp.maximum(aa_ref[...] * dis + b_ref[:, :DW], 0.0)
    hb = jnp.maximum(ab_ref[...] * dis + b_ref[:, DW:], 0.0)
    h = jnp.concatenate([ha, hb], axis=1)
    r = jnp.dot(h, w_ref[...], preferred_element_type=jnp.float32) * dis
    oa_ref[...] = r[:, :DW]
    ob_ref[...] = r[:, DW:]


def _epi_body(aa_ref, ab_ref, deg_ref, b_ref, o_ref):
    dis = lax.rsqrt(deg_ref[...] + 1.0)
    o_ref[:, :DW] = aa_ref[...] * dis + b_ref[:, :DW]
    o_ref[:, DW:] = ab_ref[...] * dis + b_ref[:, DW:]


_ROWS_BLK = 256
_GRID = (N_PAD // _ROWS_BLK,)
_xspec = pl.BlockSpec((_ROWS_BLK, D), lambda i: (i, 0))
_hspec = pl.BlockSpec((_ROWS_BLK, DW), lambda i: (i, 0))
_wspec = pl.BlockSpec((D, D), lambda i: (0, 0))
_dspec = pl.BlockSpec((_ROWS_BLK, 1), lambda i: (i, 0))
_bspec = pl.BlockSpec((1, D), lambda i: (0, 0))
_oshape = jax.ShapeDtypeStruct((N_PAD, D), jnp.float32)
_hshape = jax.ShapeDtypeStruct((N_PAD, DW), jnp.float32)

_mm_first = pl.pallas_call(
    _mm_first_body, grid=_GRID,
    in_specs=[_xspec, _wspec, _dspec],
    out_specs=[_hspec, _hspec], out_shape=[_hshape, _hshape])

_mm_mid = pl.pallas_call(
    _mm_mid_body, grid=_GRID,
    in_specs=[_hspec, _hspec, _wspec, _dspec, _bspec],
    out_specs=[_hspec, _hspec], out_shape=[_hshape, _hshape])

_epi = pl.pallas_call(
    _epi_body, grid=_GRID,
    in_specs=[_hspec, _hspec, _dspec, _bspec],
    out_specs=_xspec, out_shape=_oshape)


def kernel(x, edge_index, W1, b1, W2, b2, W3, b3):
    src = edge_index[0]
    dst = edge_index[1]
    pad = E_PAD - E
    src_p = jnp.concatenate([src, jnp.zeros((pad,), src.dtype)])
    dst_p = jnp.concatenate([dst, jnp.full((pad,), N_PAD, dst.dtype)])
    x_p = jnp.pad(x, ((0, N_PAD - N), (0, 0)))

    deg = _deg_kernel(dst_p)[:, :1]
    b1r = b1.reshape(1, D)
    b2r = b2.reshape(1, D)
    b3r = b3.reshape(1, D)

    ha, hb = _mm_first(x_p, W1, deg)
    aa, ab = _agg_kernel(ha, hb, src_p, dst_p)
    ha, hb = _mm_mid(aa, ab, W2, deg, b1r)
    aa, ab = _agg_kernel(ha, hb, src_p, dst_p)
    ha, hb = _mm_mid(aa, ab, W3, deg, b2r)
    aa, ab = _agg_kernel(ha, hb, src_p, dst_p)
    out = _epi(aa, ab, deg, b3r)
    return out[:N]


# speedup vs baseline: 1.8581x; 1.6938x over previous
import functools

import jax
import jax.numpy as jnp
from jax import lax
from jax.experimental import pallas as pl
from jax.experimental.pallas import tpu as pltpu
from jax.experimental.pallas import tpu_sc as plsc

N = 10000
E = 160000
D = 256
N_PAD = 10240
N_PAD2 = 10496
E_PAD = 163840
NHALF = N_PAD // 2
EHALF = E_PAD // 2
CHUNK = 128
EDGES_PER_TILE = EHALF // 16
CHUNKS_PER_TILE = EDGES_PER_TILE // CHUNK
ROWS_PER_TILE = N_PAD // 16
ZBLK = 64
DEG_W = 8
DEG_EPT = E_PAD // 16
DEG_CHUNKS = DEG_EPT // CHUNK
DEG_ROWS = NHALF // 16
DEG_ACC_ROWS = NHALF + 8

_mesh = plsc.VectorSubcoreMesh(core_axis_name="c", subcore_axis_name="s")


@functools.partial(
    pl.kernel,
    out_type=jax.ShapeDtypeStruct((N_PAD, DEG_W), jnp.float32),
    mesh=_mesh,
    scratch_types=[
        pltpu.VMEM((DEG_EPT,), jnp.int32),
        pltpu.VMEM((DEG_CHUNKS, CHUNK), jnp.int32),
        pltpu.VMEM((CHUNK, DEG_W), jnp.float32),
        pltpu.VMEM((DEG_ROWS, DEG_W), jnp.float32),
        pltpu.VMEM_SHARED((DEG_ACC_ROWS, DEG_W), jnp.float32),
    ],
)
def _deg_kernel(dst_hbm, deg_hbm, dst_v, loc_v, ones_v, zero_v, acc):
    c = lax.axis_index("c")
    s = lax.axis_index("s")
    node_base = c * NHALF
    edge_base = s * DEG_EPT

    pltpu.sync_copy(dst_hbm.at[pl.ds(edge_base, DEG_EPT)], dst_v)

    @pl.loop(0, CHUNK)
    def _ones(i):
        ones_v[i] = jnp.full((DEG_W,), 1.0, jnp.float32)

    @pl.loop(0, DEG_ROWS)
    def _zeros(i):
        zero_v[i] = jnp.zeros((DEG_W,), jnp.float32)

    pltpu.sync_copy(zero_v, acc.at[pl.ds(s * DEG_ROWS, DEG_ROWS)])

    @pl.loop(0, DEG_CHUNKS)
    def _idx(j):
        @pl.loop(0, CHUNK // 16)
        def _idx16(i):
            v = dst_v[pl.ds(j * CHUNK + i * 16, 16)]
            loc = v - node_base
            ok = (loc >= 0) & (loc < NHALF)
            loc_v[j, pl.ds(i * 16, 16)] = jnp.where(ok, loc, NHALF)

    plsc.subcore_barrier()

    @pl.loop(0, DEG_CHUNKS)
    def _hist(j):
        pltpu.sync_copy(ones_v, acc.at[loc_v.at[j]], add=True)

    plsc.subcore_barrier()

    pltpu.sync_copy(
        acc.at[pl.ds(s * DEG_ROWS, DEG_ROWS)],
        deg_hbm.at[pl.ds(node_base + s * DEG_ROWS, DEG_ROWS)])


@functools.partial(
    pl.kernel,
    out_type=jax.ShapeDtypeStruct((2 * N_PAD2, D), jnp.float32),
    mesh=_mesh,
    scratch_types=[
        pltpu.VMEM((EDGES_PER_TILE,), jnp.int32),
        pltpu.VMEM((EDGES_PER_TILE,), jnp.int32),
        pltpu.VMEM((CHUNKS_PER_TILE, CHUNK), jnp.int32),
        pltpu.VMEM((CHUNK, D), jnp.float32),
        pltpu.VMEM((CHUNK, D), jnp.float32),
        pltpu.VMEM((ZBLK, D), jnp.float32),
        pltpu.SemaphoreType.DMA,
        pltpu.SemaphoreType.DMA,
        pltpu.SemaphoreType.DMA,
        pltpu.SemaphoreType.DMA,
    ],
)
def _agg_kernel(h_hbm, src_hbm, dst_hbm, o_hbm,
                src_v, dst_v, loc_v, msg0, msg1, zeros_v,
                g0, g1, s0, s1):
    c = lax.axis_index("c")
    s = lax.axis_index("s")
    edge_base = c * EHALF + s * EDGES_PER_TILE
    half_base = c * N_PAD2
    row0 = s * ROWS_PER_TILE

    pltpu.sync_copy(src_hbm.at[pl.ds(edge_base, EDGES_PER_TILE)], src_v)
    pltpu.sync_copy(dst_hbm.at[pl.ds(edge_base, EDGES_PER_TILE)], dst_v)

    @pl.loop(0, CHUNKS_PER_TILE)
    def _idx(j):
        @pl.loop(0, CHUNK // 16)
        def _idx16(i):
            v = dst_v[pl.ds(j * CHUNK + i * 16, 16)]
            loc_v[j, pl.ds(i * 16, 16)] = v + half_base

    @pl.when(c == 0)
    def _init_h():
        pltpu.sync_copy(
            h_hbm.at[pl.ds(row0, ROWS_PER_TILE)],
            o_hbm.at[pl.ds(row0, ROWS_PER_TILE)])

    @pl.when(c == 1)
    def _init_zero():
        @pl.loop(0, ZBLK)
        def _z(i):
            zeros_v[i] = jnp.zeros((D,), jnp.float32)

        @pl.loop(0, ROWS_PER_TILE // ZBLK)
        def _zi(i):
            pltpu.sync_copy(
                zeros_v, o_hbm.at[pl.ds(N_PAD2 + row0 + i * ZBLK, ZBLK)])

    plsc.subcore_barrier()

    NBUF = 2
    bufs = (msg0, msg1)
    gsems = (g0, g1)
    ssems = (s0, s1)

    def _gather(j, buf, sem):
        return pltpu.async_copy(
            h_hbm.at[src_v.at[pl.ds(j * CHUNK, CHUNK)]], buf, sem)

    def _wait_gather(j, buf, sem):
        pltpu.make_async_copy(
            h_hbm.at[src_v.at[pl.ds(j * CHUNK, CHUNK)]], buf, sem).wait()

    def _scatter(j, buf, sem):
        pltpu.async_copy(buf, o_hbm.at[loc_v.at[j]], sem, add=True)

    def _wait_scatter(j, buf, sem):
        pltpu.make_async_copy(buf, o_hbm.at[loc_v.at[j]], sem).wait()

    for b in range(NBUF):
        _gather(b, bufs[b], gsems[b])

    @pl.loop(0, CHUNKS_PER_TILE, step=NBUF)
    def _main(j):
        for b in range(NBUF):
            k = j + b
            _wait_gather(k, bufs[b], gsems[b])
            _scatter(k, bufs[b], ssems[b])

        for b in range(NBUF):
            k = j + b

            @pl.when(k + NBUF < CHUNKS_PER_TILE)
            def _next():
                _wait_scatter(k, bufs[b], ssems[b])
                _gather(k + NBUF, bufs[b], gsems[b])

    for b in range(NBUF):
        _wait_scatter(CHUNKS_PER_TILE - NBUF + b, bufs[b], ssems[b])


def _mm_first_body(x_ref, w_ref, deg_ref, o_ref):
    dis = lax.rsqrt(deg_ref[...] + 1.0)
    o_ref[...] = jnp.dot(x_ref[...], w_ref[...],
                         preferred_element_type=jnp.float32) * dis


def _mm_mid_body(a0_ref, a1_ref, w_ref, deg_ref, b_ref, o_ref):
    dis = lax.rsqrt(deg_ref[...] + 1.0)
    h = jnp.maximum(
        (a0_ref[...] + a1_ref[...]) * dis + b_ref[...], 0.0)
    o_ref[...] = jnp.dot(h, w_ref[...],
                         preferred_element_type=jnp.float32) * dis


def _epi_body(a0_ref, a1_ref, deg_ref, b_ref, o_ref):
    dis = lax.rsqrt(deg_ref[...] + 1.0)
    o_ref[...] = (a0_ref[...] + a1_ref[...]) * dis + b_ref[...]


_ROWS_BLK = 256
_GRID = (N_PAD // _ROWS_BLK,)
_HALF_BLKS = N_PAD2 // _ROWS_BLK
_xspec = pl.BlockSpec((_ROWS_BLK, D), lambda i: (i, 0))
_a0spec = pl.BlockSpec((_ROWS_BLK, D), lambda i: (i, 0))
_a1spec = pl.BlockSpec((_ROWS_BLK, D), lambda i: (i + _HALF_BLKS, 0))
_wspec = pl.BlockSpec((D, D), lambda i: (0, 0))
_dspec = pl.BlockSpec((_ROWS_BLK, 1), lambda i: (i, 0))
_bspec = pl.BlockSpec((1, D), lambda i: (0, 0))
_oshape = jax.ShapeDtypeStruct((N_PAD, D), jnp.float32)

_mm_first = pl.pallas_call(
    _mm_first_body, grid=_GRID,
    in_specs=[_xspec, _wspec, _dspec],
    out_specs=_xspec, out_shape=_oshape)

_mm_mid = pl.pallas_call(
    _mm_mid_body, grid=_GRID,
    in_specs=[_a0spec, _a1spec, _wspec, _dspec, _bspec],
    out_specs=_xspec, out_shape=_oshape)

_epi = pl.pallas_call(
    _epi_body, grid=_GRID,
    in_specs=[_a0spec, _a1spec, _dspec, _bspec],
    out_specs=_xspec, out_shape=_oshape)


def kernel(x, edge_index, W1, b1, W2, b2, W3, b3):
    src = edge_index[0]
    dst = edge_index[1]
    pad = E_PAD - E
    src_p = jnp.concatenate([src, jnp.zeros((pad,), src.dtype)])
    dst_p = jnp.concatenate(
        [dst, N_PAD + (jnp.arange(pad, dtype=dst.dtype) % 256)])
    x_p = jnp.pad(x, ((0, N_PAD - N), (0, 0)))

    deg = _deg_kernel(dst_p)[:, :1]
    b1r = b1.reshape(1, D)
    b2r = b2.reshape(1, D)
    b3r = b3.reshape(1, D)

    h = _mm_first(x_p, W1, deg)
    a = _agg_kernel(h, src_p, dst_p)
    h = _mm_mid(a, a, W2, deg, b1r)
    a = _agg_kernel(h, src_p, dst_p)
    h = _mm_mid(a, a, W3, deg, b2r)
    a = _agg_kernel(h, src_p, dst_p)
    out = _epi(a, a, deg, b3r)
    return out[:N]
